# baseline (speedup 1.0000x reference)
import functools

import jax
import jax.numpy as jnp
from jax import lax
from jax.experimental import pallas as pl
from jax.experimental.pallas import tpu as pltpu
from jax.experimental.pallas import tpu_sc as plsc

_V = 1000000
_D = 64
_B = 4096
_L = 50
_HID = 100
_NCLS = 4
_HPAD = 128

_NC = 2
_NS = 16
_NW = _NC * _NS
_BPW = _B // _NW
_EPG = 2
_GROUP = _EPG * _L
_NG = _BPW // _EPG


def _pool_sc(xg, xp, tbl2):
    mesh = plsc.VectorSubcoreMesh(core_axis_name="c", subcore_axis_name="s")

    @functools.partial(
        pl.kernel,
        out_type=jax.ShapeDtypeStruct((_B, _D), jnp.float32),
        mesh=mesh,
        compiler_params=pltpu.CompilerParams(use_tc_tiling_on_sc=False),
        scratch_types=[
            pltpu.VMEM((_NG, _GROUP), jnp.int32),
            pltpu.VMEM((_GROUP, 2 * _D), jnp.float32),
            pltpu.VMEM((_GROUP, 2 * _D), jnp.float32),
            pltpu.VMEM((_BPW, _D), jnp.float32),
            pltpu.VMEM((_GROUP, 16), jnp.float32),
            pltpu.VMEM((_GROUP, 16), jnp.float32),
            pltpu.SemaphoreType.DMA,
            pltpu.SemaphoreType.DMA,
            pltpu.SemaphoreType.DMA,
            pltpu.SemaphoreType.DMA,
        ],
    )
    def k(xg_hbm, xp_hbm, tbl_hbm, out_hbm,
          idx_v, rows0, rows1, pooled_v, pe0, pe1,
          sem0, sem1, psem0, psem1):
        wid = lax.axis_index("s") * _NC + lax.axis_index("c")
        pltpu.sync_copy(xg_hbm.at[wid], idx_v)

        def fire(g, rows, sem, pe, psem):
            pltpu.async_copy(tbl_hbm.at[idx_v.at[g]], rows, sem)
            pltpu.async_copy(xp_hbm.at[wid, g], pe, psem)

        def wait(g, rows, sem, pe, psem):
            pltpu.make_async_copy(tbl_hbm.at[idx_v.at[g]], rows, sem).wait()
            pltpu.make_async_copy(xp_hbm.at[wid, g], pe, psem).wait()

        def accum(g, rows, pe):
            def body(r, accs):
                nxt = []
                for e in range(_EPG):
                    row = e * _L + r
                    p = pe[row, :]
                    for d4 in range(4):
                        lo = rows[row, pl.ds(16 * d4, 16)]
                        hi = rows[row, pl.ds(_D + 16 * d4, 16)]
                        nxt.append(accs[e * 4 + d4] + lo + p * (hi - lo))
                return tuple(nxt)

            init = tuple(jnp.zeros((16,), jnp.float32)
                         for _ in range(_EPG * 4))
            accs = lax.fori_loop(0, _L, body, init, unroll=2)
            for e in range(_EPG):
                for d4 in range(4):
                    pooled_v[g * _EPG + e, pl.ds(16 * d4, 16)] = accs[e * 4 + d4]

        fire(0, rows0, sem0, pe0, psem0)
        fire(1, rows1, sem1, pe1, psem1)

        def gbody(i, _):
            g0 = 2 * i
            wait(g0, rows0, sem0, pe0, psem0)
            accum(g0, rows0, pe0)

            @pl.when(g0 + 2 < _NG)
            def _f0():
                fire(g0 + 2, rows0, sem0, pe0, psem0)

            wait(g0 + 1, rows1, sem1, pe1, psem1)
            accum(g0 + 1, rows1, pe1)

            @pl.when(g0 + 3 < _NG)
            def _f1():
                fire(g0 + 3, rows1, sem1, pe1, psem1)

            return 0

        lax.fori_loop(0, _NG // 2, gbody, 0)
        pltpu.sync_copy(pooled_v, out_hbm.at[pl.ds(wid * _BPW, _BPW)])

    return k(xg, xp, tbl2)


def _mlp_tc(pooled, w1p, b1p, w2p, b2p):
    def body(p_ref, w1_ref, b1_ref, w2_ref, b2_ref, o_ref):
        h = jnp.dot(p_ref[...], w1_ref[...],
                    preferred_element_type=jnp.float32)
        h = jnp.maximum(h + b1_ref[...], 0.0)
        o = jnp.dot(h, w2_ref[...], preferred_element_type=jnp.float32)
        o_ref[...] = jnp.maximum(o + b2_ref[...], 0.0)

    return pl.pallas_call(
        body,
        out_shape=jax.ShapeDtypeStruct((_B, _NCLS), jnp.float32),
    )(pooled, w1p, b1p, w2p, b2p)


def kernel(x, table, W1, b1, W2, b2):
    xg = (x >> 1).reshape(_NW, _NG, _GROUP)
    xp = jnp.broadcast_to(
        (x & 1).astype(jnp.float32).reshape(_NW, _NG, _GROUP, 1),
        (_NW, _NG, _GROUP, 16))
    tbl2 = table.reshape(_V // 2, 2 * _D)
    pooled = _pool_sc(xg, xp, tbl2)
    w1p = jnp.pad(W1, ((0, 0), (0, _HPAD - _HID)))
    b1p = jnp.pad(b1, (0, _HPAD - _HID)).reshape(1, _HPAD)
    w2p = jnp.pad(W2, ((0, _HPAD - _HID), (0, 0)))
    b2p = b2.reshape(1, _NCLS)
    return _mlp_tc(pooled, w1p, b1p, w2p, b2p)


# speedup vs baseline: 1.5982x; 1.5982x over previous
import functools

import jax
import jax.numpy as jnp
from jax import lax
from jax.experimental import pallas as pl
from jax.experimental.pallas import tpu as pltpu
from jax.experimental.pallas import tpu_sc as plsc

_V = 1000000
_D = 64
_B = 4096
_L = 50
_HID = 100
_NCLS = 4
_HPAD = 128

_NC = 2
_NS = 16
_NW = _NC * _NS
_BPW = _B // _NW
_EPG = 2
_GROUP = _EPG * _L
_NG = _BPW // _EPG


_CB = 4096
_NB = (_V + _CB - 1) // _CB
_NL = _NB * (_CB // 2)


def _transpose_tc(tblT):
    def body(t_ref, o_ref):
        t = t_ref[...]
        o_ref[:, 0:_D] = t[:, 0:_CB // 2].T
        o_ref[:, _D:2 * _D] = t[:, _CB // 2:_CB].T

    return pl.pallas_call(
        body,
        grid=(_NB,),
        in_specs=[pl.BlockSpec((_D, _CB), lambda i: (0, i))],
        out_specs=pl.BlockSpec((_CB // 2, 2 * _D), lambda i: (i, 0)),
        out_shape=jax.ShapeDtypeStruct((_NL, 2 * _D), jnp.float32),
    )(tblT)


def _pool_sc(xg, xp, tbl2):
    mesh = plsc.VectorSubcoreMesh(core_axis_name="c", subcore_axis_name="s")

    @functools.partial(
        pl.kernel,
        out_type=jax.ShapeDtypeStruct((_B, _D), jnp.float32),
        mesh=mesh,
        compiler_params=pltpu.CompilerParams(use_tc_tiling_on_sc=False),
        scratch_types=[
            pltpu.VMEM((_NG, _GROUP), jnp.int32),
            pltpu.VMEM((_GROUP, 2 * _D), jnp.float32),
            pltpu.VMEM((_GROUP, 2 * _D), jnp.float32),
            pltpu.VMEM((_BPW, _D), jnp.float32),
            pltpu.VMEM((_GROUP * 16,), jnp.float32),
            pltpu.VMEM((_GROUP * 16,), jnp.float32),
            pltpu.SemaphoreType.DMA,
            pltpu.SemaphoreType.DMA,
            pltpu.SemaphoreType.DMA,
            pltpu.SemaphoreType.DMA,
        ],
    )
    def k(xg_hbm, xp_hbm, tbl_hbm, out_hbm,
          idx_v, rows0, rows1, pooled_v, pe0, pe1,
          sem0, sem1, psem0, psem1):
        wid = lax.axis_index("s") * _NC + lax.axis_index("c")
        pltpu.sync_copy(xg_hbm.at[wid], idx_v)

        def fire(g, rows, sem, pe, psem):
            pltpu.async_copy(tbl_hbm.at[idx_v.at[g]], rows, sem)
            pltpu.async_copy(
                xp_hbm.at[wid, pl.ds(g * _GROUP * 16, _GROUP * 16)], pe, psem)

        def wait(g, rows, sem, pe, psem):
            pltpu.make_async_copy(tbl_hbm.at[idx_v.at[g]], rows, sem).wait()
            pltpu.make_async_copy(
                xp_hbm.at[wid, pl.ds(g * _GROUP * 16, _GROUP * 16)],
                pe, psem).wait()

        def accum(g, rows, pe):
            def body(r, accs):
                nxt = []
                for e in range(_EPG):
                    row = e * _L + r
                    p = pe[pl.ds(row * 16, 16)]
                    for d4 in range(4):
                        lo = rows[row, pl.ds(16 * d4, 16)]
                        hi = rows[row, pl.ds(_D + 16 * d4, 16)]
                        nxt.append(accs[e * 4 + d4] + lo + p * (hi - lo))
                return tuple(nxt)

            init = tuple(jnp.zeros((16,), jnp.float32)
                         for _ in range(_EPG * 4))
            accs = lax.fori_loop(0, _L, body, init, unroll=2)
            for e in range(_EPG):
                for d4 in range(4):
                    pooled_v[g * _EPG + e, pl.ds(16 * d4, 16)] = accs[e * 4 + d4]

        fire(0, rows0, sem0, pe0, psem0)
        fire(1, rows1, sem1, pe1, psem1)

        def gbody(i, _):
            g0 = 2 * i
            wait(g0, rows0, sem0, pe0, psem0)
            accum(g0, rows0, pe0)

            @pl.when(g0 + 2 < _NG)
            def _f0():
                fire(g0 + 2, rows0, sem0, pe0, psem0)

            wait(g0 + 1, rows1, sem1, pe1, psem1)
            accum(g0 + 1, rows1, pe1)

            @pl.when(g0 + 3 < _NG)
            def _f1():
                fire(g0 + 3, rows1, sem1, pe1, psem1)

            return 0

        lax.fori_loop(0, _NG // 2, gbody, 0)
        pltpu.sync_copy(pooled_v, out_hbm.at[pl.ds(wid * _BPW, _BPW)])

    return k(xg, xp, tbl2)


def _mlp_tc(pooled, w1p, b1p, w2p, b2p):
    def body(p_ref, w1_ref, b1_ref, w2_ref, b2_ref, o_ref):
        h = jnp.dot(p_ref[...], w1_ref[...],
                    preferred_element_type=jnp.float32)
        h = jnp.maximum(h + b1_ref[...], 0.0)
        o = jnp.dot(h, w2_ref[...], preferred_element_type=jnp.float32)
        o_ref[...] = jnp.maximum(o + b2_ref[...], 0.0)

    return pl.pallas_call(
        body,
        out_shape=jax.ShapeDtypeStruct((_B, _NCLS), jnp.float32),
    )(pooled, w1p, b1p, w2p, b2p)


def kernel(x, table, W1, b1, W2, b2):
    xg = ((x >> 12) * (_CB // 2) + (x & (_CB // 2 - 1))
          ).reshape(_NW, _NG, _GROUP)
    xp = jnp.broadcast_to(
        ((x >> 11) & 1).astype(jnp.float32).reshape(_NW, _NG * _GROUP, 1),
        (_NW, _NG * _GROUP, 16)).reshape(_NW, _NG * _GROUP * 16)
    tbl2 = _transpose_tc(table.T)
    pooled = _pool_sc(xg, xp, tbl2)
    w1p = jnp.pad(W1, ((0, 0), (0, _HPAD - _HID)))
    b1p = jnp.pad(b1, (0, _HPAD - _HID)).reshape(1, _HPAD)
    w2p = jnp.pad(W2, ((0, _HPAD - _HID), (0, 0)))
    b2p = b2.reshape(1, _NCLS)
    return _mlp_tc(pooled, w1p, b1p, w2p, b2p)


# speedup vs baseline: 2.0219x; 1.2651x over previous
import functools

import jax
import jax.numpy as jnp
from jax import lax
from jax.experimental import pallas as pl
from jax.experimental.pallas import tpu as pltpu
from jax.experimental.pallas import tpu_sc as plsc

_V = 1000000
_D = 64
_B = 4096
_L = 50
_HID = 100
_NCLS = 4
_HPAD = 128

_NC = 2
_NS = 16
_NW = _NC * _NS
_BPW = _B // _NW
_EPG = 2
_GROUP = _EPG * _L
_NG = _BPW // _EPG


_CB = 8192
_NB = (_V + _CB - 1) // _CB
_NL = _NB * (_CB // 2)


def _transpose_tc(tblT):
    def body(t_ref, o_ref):
        t = t_ref[...]
        o_ref[:, 0:_D] = t[:, 0:_CB // 2].T
        o_ref[:, _D:2 * _D] = t[:, _CB // 2:_CB].T

    return pl.pallas_call(
        body,
        grid=(_NB,),
        in_specs=[pl.BlockSpec((_D, _CB), lambda i: (0, i))],
        out_specs=pl.BlockSpec((_CB // 2, 2 * _D), lambda i: (i, 0)),
        out_shape=jax.ShapeDtypeStruct((_NL, 2 * _D), jnp.float32),
    )(tblT)


def _pool_sc(xg, xp, tbl2):
    mesh = plsc.VectorSubcoreMesh(core_axis_name="c", subcore_axis_name="s")

    @functools.partial(
        pl.kernel,
        out_type=jax.ShapeDtypeStruct((_B, _D), jnp.float32),
        mesh=mesh,
        compiler_params=pltpu.CompilerParams(use_tc_tiling_on_sc=False),
        scratch_types=[
            pltpu.VMEM((_NG, _GROUP), jnp.int32),
            pltpu.VMEM((_GROUP, 2 * _D), jnp.float32),
            pltpu.VMEM((_GROUP, 2 * _D), jnp.float32),
            pltpu.VMEM((_BPW, _D), jnp.float32),
            pltpu.VMEM((_GROUP * 16,), jnp.float32),
            pltpu.VMEM((_GROUP * 16,), jnp.float32),
            pltpu.SemaphoreType.DMA,
            pltpu.SemaphoreType.DMA,
            pltpu.SemaphoreType.DMA,
            pltpu.SemaphoreType.DMA,
        ],
    )
    def k(xg_hbm, xp_hbm, tbl_hbm, out_hbm,
          idx_v, rows0, rows1, pooled_v, pe0, pe1,
          sem0, sem1, psem0, psem1):
        wid = lax.axis_index("s") * _NC + lax.axis_index("c")
        pltpu.sync_copy(xg_hbm.at[wid], idx_v)

        def fire(g, rows, sem, pe, psem):
            pltpu.async_copy(tbl_hbm.at[idx_v.at[g]], rows, sem)
            pltpu.async_copy(
                xp_hbm.at[wid, pl.ds(g * _GROUP * 16, _GROUP * 16)], pe, psem)

        def wait(g, rows, sem, pe, psem):
            pltpu.make_async_copy(tbl_hbm.at[idx_v.at[g]], rows, sem).wait()
            pltpu.make_async_copy(
                xp_hbm.at[wid, pl.ds(g * _GROUP * 16, _GROUP * 16)],
                pe, psem).wait()

        def accum(g, rows, pe):
            def body(r, accs):
                nxt = []
                for e in range(_EPG):
                    row = e * _L + r
                    p = pe[pl.ds(row * 16, 16)]
                    for d4 in range(4):
                        lo = rows[row, pl.ds(16 * d4, 16)]
                        hi = rows[row, pl.ds(_D + 16 * d4, 16)]
                        nxt.append(accs[e * 4 + d4] + lo + p * (hi - lo))
                return tuple(nxt)

            init = tuple(jnp.zeros((16,), jnp.float32)
                         for _ in range(_EPG * 4))
            accs = lax.fori_loop(0, _L, body, init, unroll=2)
            for e in range(_EPG):
                for d4 in range(4):
                    pooled_v[g * _EPG + e, pl.ds(16 * d4, 16)] = accs[e * 4 + d4]

        fire(0, rows0, sem0, pe0, psem0)
        fire(1, rows1, sem1, pe1, psem1)

        def gbody(i, _):
            g0 = 2 * i
            wait(g0, rows0, sem0, pe0, psem0)
            accum(g0, rows0, pe0)

            @pl.when(g0 + 2 < _NG)
            def _f0():
                fire(g0 + 2, rows0, sem0, pe0, psem0)

            wait(g0 + 1, rows1, sem1, pe1, psem1)
            accum(g0 + 1, rows1, pe1)

            @pl.when(g0 + 3 < _NG)
            def _f1():
                fire(g0 + 3, rows1, sem1, pe1, psem1)

            return 0

        lax.fori_loop(0, _NG // 2, gbody, 0)
        pltpu.sync_copy(pooled_v, out_hbm.at[pl.ds(wid * _BPW, _BPW)])

    return k(xg, xp, tbl2)


def _mlp_tc(pooled, w1p, b1p, w2p, b2p):
    def body(p_ref, w1_ref, b1_ref, w2_ref, b2_ref, o_ref):
        h = jnp.dot(p_ref[...], w1_ref[...],
                    preferred_element_type=jnp.float32)
        h = jnp.maximum(h + b1_ref[...], 0.0)
        o = jnp.dot(h, w2_ref[...], preferred_element_type=jnp.float32)
        o_ref[...] = jnp.maximum(o + b2_ref[...], 0.0)

    return pl.pallas_call(
        body,
        out_shape=jax.ShapeDtypeStruct((_B, _NCLS), jnp.float32),
    )(pooled, w1p, b1p, w2p, b2p)


def kernel(x, table, W1, b1, W2, b2):
    xg = ((x // _CB) * (_CB // 2) + (x & (_CB // 2 - 1))
          ).reshape(_NW, _NG, _GROUP)
    hi = ((x // (_CB // 2)) & 1).astype(jnp.float32).reshape(_B * _L // 8, 8)
    sel = (jnp.arange(128)[None, :] // 16
           == jnp.arange(8)[:, None]).astype(jnp.float32)
    xp = jnp.dot(hi, sel).reshape(_NW, _NG * _GROUP * 16)
    tbl2 = _transpose_tc(table.T)
    pooled = _pool_sc(xg, xp, tbl2)
    w1p = jnp.pad(W1, ((0, 0), (0, _HPAD - _HID)))
    b1p = jnp.pad(b1, (0, _HPAD - _HID)).reshape(1, _HPAD)
    w2p = jnp.pad(W2, ((0, _HPAD - _HID), (0, 0)))
    b2p = b2.reshape(1, _NCLS)
    return _mlp_tc(pooled, w1p, b1p, w2p, b2p)


# speedup vs baseline: 2.2091x; 1.0926x over previous
import functools

import jax
import jax.numpy as jnp
from jax import lax
from jax.experimental import pallas as pl
from jax.experimental.pallas import tpu as pltpu
from jax.experimental.pallas import tpu_sc as plsc

_V = 1000000
_D = 64
_B = 4096
_L = 50
_HID = 100
_NCLS = 4
_HPAD = 128

_NC = 2
_NS = 16
_NW = _NC * _NS
_BPW = _B // _NW
_EPG = 2
_GROUP = _EPG * _L
_NG = _BPW // _EPG


_CB = 16384
_NB = (_V + _CB - 1) // _CB
_NL = _NB * (_CB // 2)


def _transpose_tc(tblT):
    def body(t_ref, o_ref):
        t = t_ref[...]
        o_ref[:, 0:_D] = t[:, 0:_CB // 2].T
        o_ref[:, _D:2 * _D] = t[:, _CB // 2:_CB].T

    return pl.pallas_call(
        body,
        grid=(_NB,),
        in_specs=[pl.BlockSpec((_D, _CB), lambda i: (0, i))],
        out_specs=pl.BlockSpec((_CB // 2, 2 * _D), lambda i: (i, 0)),
        out_shape=jax.ShapeDtypeStruct((_NL, 2 * _D), jnp.float32),
    )(tblT)


def _pool_sc(xg, xp, tbl2):
    mesh = plsc.VectorSubcoreMesh(core_axis_name="c", subcore_axis_name="s")

    @functools.partial(
        pl.kernel,
        out_type=jax.ShapeDtypeStruct((_B, _D), jnp.float32),
        mesh=mesh,
        compiler_params=pltpu.CompilerParams(use_tc_tiling_on_sc=False),
        scratch_types=[
            pltpu.VMEM((_NG, _GROUP), jnp.int32),
            pltpu.VMEM((_GROUP, 2 * _D), jnp.float32),
            pltpu.VMEM((_GROUP, 2 * _D), jnp.float32),
            pltpu.VMEM((_BPW, _D), jnp.float32),
            pltpu.VMEM((_GROUP * 16,), jnp.float32),
            pltpu.VMEM((_GROUP * 16,), jnp.float32),
            pltpu.SemaphoreType.DMA,
            pltpu.SemaphoreType.DMA,
            pltpu.SemaphoreType.DMA,
            pltpu.SemaphoreType.DMA,
        ],
    )
    def k(xg_hbm, xp_hbm, tbl_hbm, out_hbm,
          idx_v, rows0, rows1, pooled_v, pe0, pe1,
          sem0, sem1, psem0, psem1):
        wid = lax.axis_index("s") * _NC + lax.axis_index("c")
        pltpu.sync_copy(xg_hbm.at[wid], idx_v)

        def fire(g, rows, sem, pe, psem):
            pltpu.async_copy(tbl_hbm.at[idx_v.at[g]], rows, sem)
            pltpu.async_copy(
                xp_hbm.at[wid, pl.ds(g * _GROUP * 16, _GROUP * 16)], pe, psem)

        def wait(g, rows, sem, pe, psem):
            pltpu.make_async_copy(tbl_hbm.at[idx_v.at[g]], rows, sem).wait()
            pltpu.make_async_copy(
                xp_hbm.at[wid, pl.ds(g * _GROUP * 16, _GROUP * 16)],
                pe, psem).wait()

        def accum(g, rows, pe):
            def body(r, accs):
                nxt = []
                for e in range(_EPG):
                    row = e * _L + r
                    p = pe[pl.ds(row * 16, 16)]
                    for d4 in range(4):
                        lo = rows[row, pl.ds(16 * d4, 16)]
                        hi = rows[row, pl.ds(_D + 16 * d4, 16)]
                        nxt.append(accs[e * 4 + d4] + lo + p * (hi - lo))
                return tuple(nxt)

            init = tuple(jnp.zeros((16,), jnp.float32)
                         for _ in range(_EPG * 4))
            accs = lax.fori_loop(0, _L, body, init, unroll=2)
            for e in range(_EPG):
                for d4 in range(4):
                    pooled_v[g * _EPG + e, pl.ds(16 * d4, 16)] = accs[e * 4 + d4]

        fire(0, rows0, sem0, pe0, psem0)
        fire(1, rows1, sem1, pe1, psem1)

        def gbody(i, _):
            g0 = 2 * i
            wait(g0, rows0, sem0, pe0, psem0)
            accum(g0, rows0, pe0)

            @pl.when(g0 + 2 < _NG)
            def _f0():
                fire(g0 + 2, rows0, sem0, pe0, psem0)

            wait(g0 + 1, rows1, sem1, pe1, psem1)
            accum(g0 + 1, rows1, pe1)

            @pl.when(g0 + 3 < _NG)
            def _f1():
                fire(g0 + 3, rows1, sem1, pe1, psem1)

            return 0

        lax.fori_loop(0, _NG // 2, gbody, 0)
        pltpu.sync_copy(pooled_v, out_hbm.at[pl.ds(wid * _BPW, _BPW)])

    return k(xg, xp, tbl2)


def _mlp_tc(pooled, w1p, b1p, w2p, b2p):
    def body(p_ref, w1_ref, b1_ref, w2_ref, b2_ref, o_ref):
        h = jnp.dot(p_ref[...], w1_ref[...],
                    preferred_element_type=jnp.float32)
        h = jnp.maximum(h + b1_ref[...], 0.0)
        o = jnp.dot(h, w2_ref[...], preferred_element_type=jnp.float32)
        o_ref[...] = jnp.maximum(o + b2_ref[...], 0.0)

    return pl.pallas_call(
        body,
        out_shape=jax.ShapeDtypeStruct((_B, _NCLS), jnp.float32),
    )(pooled, w1p, b1p, w2p, b2p)


def kernel(x, table, W1, b1, W2, b2):
    xg = ((x // _CB) * (_CB // 2) + (x & (_CB // 2 - 1))
          ).reshape(_NW, _NG, _GROUP)
    hi = ((x // (_CB // 2)) & 1).astype(jnp.float32).reshape(_B * _L // 8, 8)
    sel = (jnp.arange(128)[None, :] // 16
           == jnp.arange(8)[:, None]).astype(jnp.float32)
    xp = jnp.dot(hi, sel).reshape(_NW, _NG * _GROUP * 16)
    tbl2 = _transpose_tc(table.T)
    pooled = _pool_sc(xg, xp, tbl2)
    w1p = jnp.pad(W1, ((0, 0), (0, _HPAD - _HID)))
    b1p = jnp.pad(b1, (0, _HPAD - _HID)).reshape(1, _HPAD)
    w2p = jnp.pad(W2, ((0, _HPAD - _HID), (0, 0)))
    b2p = b2.reshape(1, _NCLS)
    return _mlp_tc(pooled, w1p, b1p, w2p, b2p)


# speedup vs baseline: 2.3181x; 1.0494x over previous
import functools

import jax
import jax.numpy as jnp
from jax import lax
from jax.experimental import pallas as pl
from jax.experimental.pallas import tpu as pltpu
from jax.experimental.pallas import tpu_sc as plsc

_V = 1000000
_D = 64
_B = 4096
_L = 50
_HID = 100
_NCLS = 4
_HPAD = 128

_NC = 2
_NS = 16
_NW = _NC * _NS
_BPW = _B // _NW
_EPG = 2
_GROUP = _EPG * _L
_NG = _BPW // _EPG


_CB = 32768
_NB = (_V + _CB - 1) // _CB
_NL = _NB * (_CB // 2)


def _transpose_tc(tblT):
    def body(t_ref, o_ref):
        t = t_ref[...]
        o_ref[:, 0:_D] = t[:, 0:_CB // 2].T
        o_ref[:, _D:2 * _D] = t[:, _CB // 2:_CB].T

    return pl.pallas_call(
        body,
        grid=(_NB,),
        in_specs=[pl.BlockSpec((_D, _CB), lambda i: (0, i))],
        out_specs=pl.BlockSpec((_CB // 2, 2 * _D), lambda i: (i, 0)),
        out_shape=jax.ShapeDtypeStruct((_NL, 2 * _D), jnp.float32),
    )(tblT)


def _pool_sc(xg, xp, tbl2):
    mesh = plsc.VectorSubcoreMesh(core_axis_name="c", subcore_axis_name="s")

    @functools.partial(
        pl.kernel,
        out_type=jax.ShapeDtypeStruct((_B, _D), jnp.float32),
        mesh=mesh,
        compiler_params=pltpu.CompilerParams(use_tc_tiling_on_sc=False),
        scratch_types=[
            pltpu.VMEM((_NG, _GROUP), jnp.int32),
            pltpu.VMEM((_GROUP, 2 * _D), jnp.float32),
            pltpu.VMEM((_GROUP, 2 * _D), jnp.float32),
            pltpu.VMEM((_BPW, _D), jnp.float32),
            pltpu.VMEM((_GROUP * 16,), jnp.float32),
            pltpu.VMEM((_GROUP * 16,), jnp.float32),
            pltpu.SemaphoreType.DMA,
            pltpu.SemaphoreType.DMA,
            pltpu.SemaphoreType.DMA,
            pltpu.SemaphoreType.DMA,
        ],
    )
    def k(xg_hbm, xp_hbm, tbl_hbm, out_hbm,
          idx_v, rows0, rows1, pooled_v, pe0, pe1,
          sem0, sem1, psem0, psem1):
        wid = lax.axis_index("s") * _NC + lax.axis_index("c")
        pltpu.sync_copy(xg_hbm.at[wid], idx_v)

        def fire(g, rows, sem, pe, psem):
            pltpu.async_copy(tbl_hbm.at[idx_v.at[g]], rows, sem)
            pltpu.async_copy(
                xp_hbm.at[wid, pl.ds(g * _GROUP * 16, _GROUP * 16)], pe, psem)

        def wait(g, rows, sem, pe, psem):
            pltpu.make_async_copy(tbl_hbm.at[idx_v.at[g]], rows, sem).wait()
            pltpu.make_async_copy(
                xp_hbm.at[wid, pl.ds(g * _GROUP * 16, _GROUP * 16)],
                pe, psem).wait()

        def accum(g, rows, pe):
            def body(r, accs):
                nxt = []
                for e in range(_EPG):
                    row = e * _L + r
                    p = pe[pl.ds(row * 16, 16)]
                    for d4 in range(4):
                        lo = rows[row, pl.ds(16 * d4, 16)]
                        hi = rows[row, pl.ds(_D + 16 * d4, 16)]
                        nxt.append(accs[e * 4 + d4] + lo + p * (hi - lo))
                return tuple(nxt)

            init = tuple(jnp.zeros((16,), jnp.float32)
                         for _ in range(_EPG * 4))
            accs = lax.fori_loop(0, _L, body, init, unroll=2)
            for e in range(_EPG):
                for d4 in range(4):
                    pooled_v[g * _EPG + e, pl.ds(16 * d4, 16)] = accs[e * 4 + d4]

        fire(0, rows0, sem0, pe0, psem0)
        fire(1, rows1, sem1, pe1, psem1)

        def gbody(i, _):
            g0 = 2 * i
            wait(g0, rows0, sem0, pe0, psem0)
            accum(g0, rows0, pe0)

            @pl.when(g0 + 2 < _NG)
            def _f0():
                fire(g0 + 2, rows0, sem0, pe0, psem0)

            wait(g0 + 1, rows1, sem1, pe1, psem1)
            accum(g0 + 1, rows1, pe1)

            @pl.when(g0 + 3 < _NG)
            def _f1():
                fire(g0 + 3, rows1, sem1, pe1, psem1)

            return 0

        lax.fori_loop(0, _NG // 2, gbody, 0)
        pltpu.sync_copy(pooled_v, out_hbm.at[pl.ds(wid * _BPW, _BPW)])

    return k(xg, xp, tbl2)


def _mlp_tc(pooled, w1p, b1p, w2p, b2p):
    def body(p_ref, w1_ref, b1_ref, w2_ref, b2_ref, o_ref):
        h = jnp.dot(p_ref[...], w1_ref[...],
                    preferred_element_type=jnp.float32)
        h = jnp.maximum(h + b1_ref[...], 0.0)
        o = jnp.dot(h, w2_ref[...], preferred_element_type=jnp.float32)
        o_ref[...] = jnp.maximum(o + b2_ref[...], 0.0)

    return pl.pallas_call(
        body,
        out_shape=jax.ShapeDtypeStruct((_B, _NCLS), jnp.float32),
    )(pooled, w1p, b1p, w2p, b2p)


def kernel(x, table, W1, b1, W2, b2):
    xg = ((x // _CB) * (_CB // 2) + (x & (_CB // 2 - 1))
          ).reshape(_NW, _NG, _GROUP)
    hi = ((x // (_CB // 2)) & 1).astype(jnp.float32).reshape(_B * _L // 8, 8)
    sel = (jnp.arange(128)[None, :] // 16
           == jnp.arange(8)[:, None]).astype(jnp.float32)
    xp = jnp.dot(hi, sel).reshape(_NW, _NG * _GROUP * 16)
    tbl2 = _transpose_tc(table.T)
    pooled = _pool_sc(xg, xp, tbl2)
    w1p = jnp.pad(W1, ((0, 0), (0, _HPAD - _HID)))
    b1p = jnp.pad(b1, (0, _HPAD - _HID)).reshape(1, _HPAD)
    w2p = jnp.pad(W2, ((0, _HPAD - _HID), (0, 0)))
    b2p = b2.reshape(1, _NCLS)
    return _mlp_tc(pooled, w1p, b1p, w2p, b2p)


# speedup vs baseline: 2.3589x; 1.0176x over previous
import functools

import jax
import jax.numpy as jnp
from jax import lax
from jax.experimental import pallas as pl
from jax.experimental.pallas import tpu as pltpu
from jax.experimental.pallas import tpu_sc as plsc

_V = 1000000
_D = 64
_B = 4096
_L = 50
_HID = 100
_NCLS = 4
_HPAD = 128

_NC = 2
_NS = 16
_NW = _NC * _NS
_BPW = _B // _NW
_EPG = 2
_GROUP = _EPG * _L
_NG = _BPW // _EPG


_CB = 32768
_NB = (_V + _CB - 1) // _CB
_NL = _NB * (_CB // 2)


def _transpose_tc(tblT, hi, sel):
    hb = 1024
    nhb = _B * _L // 8 // hb

    def body(t_ref, hi_ref, sel_ref, o_ref, xp_ref):
        t = t_ref[...]
        o_ref[:, 0:_D] = t[:, 0:_CB // 2].T
        o_ref[:, _D:2 * _D] = t[:, _CB // 2:_CB].T
        xp_ref[...] = jnp.dot(hi_ref[...], sel_ref[...],
                              preferred_element_type=jnp.float32)

    return pl.pallas_call(
        body,
        grid=(_NB,),
        in_specs=[
            pl.BlockSpec((_D, _CB), lambda i: (0, i)),
            pl.BlockSpec((hb, 8), lambda i: (jnp.minimum(i, nhb - 1), 0)),
            pl.BlockSpec((8, 2 * _D), lambda i: (0, 0)),
        ],
        out_specs=[
            pl.BlockSpec((_CB // 2, 2 * _D), lambda i: (i, 0)),
            pl.BlockSpec((hb, 2 * _D), lambda i: (jnp.minimum(i, nhb - 1), 0)),
        ],
        out_shape=[
            jax.ShapeDtypeStruct((_NL, 2 * _D), jnp.float32),
            jax.ShapeDtypeStruct((_B * _L // 8, 2 * _D), jnp.float32),
        ],
    )(tblT, hi, sel)


def _pool_sc(xg, xp, tbl2):
    mesh = plsc.VectorSubcoreMesh(core_axis_name="c", subcore_axis_name="s")

    @functools.partial(
        pl.kernel,
        out_type=jax.ShapeDtypeStruct((_B, _D), jnp.float32),
        mesh=mesh,
        compiler_params=pltpu.CompilerParams(use_tc_tiling_on_sc=False),
        scratch_types=[
            pltpu.VMEM((_NG, _GROUP), jnp.int32),
            pltpu.VMEM((_GROUP, 2 * _D), jnp.float32),
            pltpu.VMEM((_GROUP, 2 * _D), jnp.float32),
            pltpu.VMEM((_BPW, _D), jnp.float32),
            pltpu.VMEM((_GROUP * 16,), jnp.float32),
            pltpu.VMEM((_GROUP * 16,), jnp.float32),
            pltpu.SemaphoreType.DMA,
            pltpu.SemaphoreType.DMA,
            pltpu.SemaphoreType.DMA,
            pltpu.SemaphoreType.DMA,
        ],
    )
    def k(xg_hbm, xp_hbm, tbl_hbm, out_hbm,
          idx_v, rows0, rows1, pooled_v, pe0, pe1,
          sem0, sem1, psem0, psem1):
        wid = lax.axis_index("s") * _NC + lax.axis_index("c")
        pltpu.sync_copy(xg_hbm.at[wid], idx_v)

        def fire(g, rows, sem, pe, psem):
            pltpu.async_copy(tbl_hbm.at[idx_v.at[g]], rows, sem)
            pltpu.async_copy(
                xp_hbm.at[wid, pl.ds(g * _GROUP * 16, _GROUP * 16)], pe, psem)

        def wait(g, rows, sem, pe, psem):
            pltpu.make_async_copy(tbl_hbm.at[idx_v.at[g]], rows, sem).wait()
            pltpu.make_async_copy(
                xp_hbm.at[wid, pl.ds(g * _GROUP * 16, _GROUP * 16)],
                pe, psem).wait()

        def accum(g, rows, pe):
            def body(r, accs):
                nxt = []
                for e in range(_EPG):
                    row = e * _L + r
                    p = pe[pl.ds(row * 16, 16)]
                    for d4 in range(4):
                        lo = rows[row, pl.ds(16 * d4, 16)]
                        hi = rows[row, pl.ds(_D + 16 * d4, 16)]
                        nxt.append(accs[e * 4 + d4] + lo + p * (hi - lo))
                return tuple(nxt)

            init = tuple(jnp.zeros((16,), jnp.float32)
                         for _ in range(_EPG * 4))
            accs = lax.fori_loop(0, _L, body, init, unroll=2)
            for e in range(_EPG):
                for d4 in range(4):
                    pooled_v[g * _EPG + e, pl.ds(16 * d4, 16)] = accs[e * 4 + d4]

        fire(0, rows0, sem0, pe0, psem0)
        fire(1, rows1, sem1, pe1, psem1)

        def gbody(i, _):
            g0 = 2 * i
            wait(g0, rows0, sem0, pe0, psem0)
            accum(g0, rows0, pe0)

            @pl.when(g0 + 2 < _NG)
            def _f0():
                fire(g0 + 2, rows0, sem0, pe0, psem0)

            wait(g0 + 1, rows1, sem1, pe1, psem1)
            accum(g0 + 1, rows1, pe1)

            @pl.when(g0 + 3 < _NG)
            def _f1():
                fire(g0 + 3, rows1, sem1, pe1, psem1)

            return 0

        lax.fori_loop(0, _NG // 2, gbody, 0)
        pltpu.sync_copy(pooled_v, out_hbm.at[pl.ds(wid * _BPW, _BPW)])

    return k(xg, xp, tbl2)


def _mlp_tc(pooled, w1p, b1p, w2p, b2p):
    def body(p_ref, w1_ref, b1_ref, w2_ref, b2_ref, o_ref):
        h = jnp.dot(p_ref[...], w1_ref[...],
                    preferred_element_type=jnp.float32)
        h = jnp.maximum(h + b1_ref[...], 0.0)
        o = jnp.dot(h, w2_ref[...], preferred_element_type=jnp.float32)
        o_ref[...] = jnp.maximum(o + b2_ref[...], 0.0)

    return pl.pallas_call(
        body,
        out_shape=jax.ShapeDtypeStruct((_B, _NCLS), jnp.float32),
    )(pooled, w1p, b1p, w2p, b2p)


def kernel(x, table, W1, b1, W2, b2):
    xg = ((x // _CB) * (_CB // 2) + (x & (_CB // 2 - 1))
          ).reshape(_NW, _NG, _GROUP)
    hi = ((x // (_CB // 2)) & 1).astype(jnp.float32).reshape(_B * _L // 8, 8)
    sel = (jnp.arange(128)[None, :] // 16
           == jnp.arange(8)[:, None]).astype(jnp.float32)
    tbl2, xp2d = _transpose_tc(table.T, hi, sel)
    xp = xp2d.reshape(_NW, _NG * _GROUP * 16)
    pooled = _pool_sc(xg, xp, tbl2)
    w1p = jnp.pad(W1, ((0, 0), (0, _HPAD - _HID)))
    b1p = jnp.pad(b1, (0, _HPAD - _HID)).reshape(1, _HPAD)
    w2p = jnp.pad(W2, ((0, _HPAD - _HID), (0, 0)))
    b2p = b2.reshape(1, _NCLS)
    return _mlp_tc(pooled, w1p, b1p, w2p, b2p)
